# baseline (speedup 1.0000x reference)
import jax
import jax.numpy as jnp
from jax import lax
from jax.experimental import pallas as pl
from jax.experimental.pallas import tpu as pltpu
from jax.experimental.pallas import tpu_sc as plsc

N = 10000
E = 320000
DH = 128
D_IN = 128
NG = 64
DOUT = 16

NC = 2
NS = 16
NW = NC * NS
EPT = E // NW
CB = 40
NCHUNK = EPT // CB
G = 5
NGRP = NCHUNK // G
W16 = 624

HHALF = 5120
NPAD = 10112
HCH = EPT // 16

BM = 5000
NBM = N // BM

_mesh = plsc.VectorSubcoreMesh(core_axis_name="c", subcore_axis_name="s",
                               num_cores=NC, num_subcores=NS)


def _deg_body(colr, out, cidx, hist, red):
    c = lax.axis_index("c")
    s = lax.axis_index("s")
    wid = c * NS + s
    pltpu.sync_copy(colr.at[pl.ds(wid * EPT, EPT)], cidx)

    lane = lax.broadcasted_iota(jnp.int32, (16,), 0)
    lanebase = lane * HHALF
    onev = jnp.full((16,), 1.0, jnp.float32)
    zerov = jnp.full((16,), 0.0, jnp.float32)

    for p in range(2):
        base = p * HHALF

        def zbody(i, carry):
            for u in range(8):
                hist[pl.ds((i * 8 + u) * 16, 16)] = zerov
            return carry

        lax.fori_loop(0, 16 * HHALF // (16 * 8), zbody, 0)

        def sbody(i, carry):
            for u in range(5):
                k = i * 5 + u
                colv = cidx[pl.ds(k * 16, 16)]
                cshift = colv - base
                m = (cshift >= 0) & (cshift < HHALF)
                plsc.addupdate_scatter(hist, [lanebase + cshift], onev,
                                       mask=m)
            return carry

        lax.fori_loop(0, HCH // 5, sbody, 0)

        nred = (HHALF if p == 0 else NPAD - HHALF) // 16

        def rbody(i, carry):
            acc = hist[pl.ds(i * 16, 16)]
            for j in range(1, 16):
                acc = acc + hist[pl.ds(j * HHALF + i * 16, 16)]
            red[pl.ds(base + i * 16, 16)] = acc
            return carry

        lax.fori_loop(0, nred, rbody, 0)

    pltpu.sync_copy(red, out.at[wid])


_deg_call = pl.kernel(
    _deg_body,
    out_type=jax.ShapeDtypeStruct((NW, NPAD), jnp.float32),
    mesh=_mesh,
    scratch_types=[
        pltpu.VMEM((EPT,), jnp.int32),
        pltpu.VMEM((16 * HHALF,), jnp.float32),
        pltpu.VMEM((NPAD,), jnp.float32),
    ],
    compiler_params=pltpu.CompilerParams(needs_layout_passes=False),
)


def _seg_body(hws, rowr, colr, zeros, out, ridx, cidx,
              r0, r1, r2, r3, r4, agg,
              g0, g1, g2, g3, g4, s0, s1, s2, s3, s4):
    rows = [r0, r1, r2, r3, r4]
    gsem = [g0, g1, g2, g3, g4]
    ssem = [s0, s1, s2, s3, s4]
    c = lax.axis_index("c")
    s = lax.axis_index("s")
    wid = c * NS + s
    pltpu.sync_copy(rowr.at[pl.ds(wid * EPT, EPT)], ridx)
    pltpu.sync_copy(colr.at[pl.ds(wid * EPT, EPT)], cidx)

    @pl.when(s < NS - 1)
    def _():
        pltpu.sync_copy(zeros.at[pl.ds(s * W16, W16)],
                        agg.at[pl.ds(s * W16, W16)])

    @pl.when(s == NS - 1)
    def _():
        pltpu.sync_copy(zeros.at[pl.ds((NS - 1) * W16, N - (NS - 1) * W16)],
                        agg.at[pl.ds((NS - 1) * W16, N - (NS - 1) * W16)])

    plsc.subcore_barrier()

    for j in range(G):
        pltpu.async_copy(hws.at[ridx.at[pl.ds(j * CB, CB)]], rows[j], gsem[j])

    def grp(g, carry):
        for j in range(G):
            k = g * G + j
            pltpu.make_async_copy(hws.at[ridx.at[pl.ds(k * CB, CB)]],
                                  rows[j], gsem[j]).wait()
            pltpu.async_copy(rows[j], agg.at[cidx.at[pl.ds(k * CB, CB)]],
                             ssem[j], add=True)

        @pl.when(g < NGRP - 1)
        def _():
            for j in range(G):
                k = g * G + j
                pltpu.make_async_copy(rows[j],
                                      agg.at[cidx.at[pl.ds(k * CB, CB)]],
                                      ssem[j]).wait()
                pltpu.async_copy(hws.at[ridx.at[pl.ds((k + G) * CB, CB)]],
                                 rows[j], gsem[j])

        return carry

    lax.fori_loop(0, NGRP, grp, 0)
    for j in range(G):
        pltpu.make_async_copy(rows[j], agg.at[cidx.at[pl.ds(0, CB)]],
                              ssem[j]).wait()
    plsc.subcore_barrier()

    @pl.when(s < NS - 1)
    def _():
        pltpu.sync_copy(agg.at[pl.ds(s * W16, W16)],
                        out.at[c, pl.ds(s * W16, W16)])

    @pl.when(s == NS - 1)
    def _():
        pltpu.sync_copy(agg.at[pl.ds((NS - 1) * W16, N - (NS - 1) * W16)],
                        out.at[c, pl.ds((NS - 1) * W16, N - (NS - 1) * W16)])


_seg_call = pl.kernel(
    _seg_body,
    out_type=jax.ShapeDtypeStruct((NC, N, DH), jnp.float32),
    mesh=_mesh,
    scratch_types=[
        pltpu.VMEM((EPT,), jnp.int32),
        pltpu.VMEM((EPT,), jnp.int32),
    ] + [pltpu.VMEM((CB, DH), jnp.float32) for _ in range(G)] + [
        pltpu.VMEM_SHARED((N, DH), jnp.float32),
    ] + [pltpu.SemaphoreType.DMA for _ in range(2 * G)],
)


def _dot(a, b):
    return lax.dot_general(
        a, b, (((1,), (0,)), ((), ())),
        preferred_element_type=jnp.float32,
        precision=lax.Precision.HIGHEST,
    )


def _projmm_body(degp_ref, x_ref, wi_ref, bi_ref, w0_ref,
                 oh_ref, ohws_ref, od_ref, d_scr):
    i = pl.program_id(0)

    @pl.when(i == 0)
    def _():
        ones = jnp.ones((NW, 1), jnp.float32)
        deg = lax.dot_general(
            degp_ref[...], ones, (((0,), (0,)), ((), ())),
            preferred_element_type=jnp.float32,
            precision=lax.Precision.HIGHEST,
        ) + 2.0
        d = lax.rsqrt(deg)
        d_scr[...] = d
        od_ref[...] = d

    h = _dot(x_ref[...], wi_ref[...]) + bi_ref[...]
    oh_ref[...] = h
    ohws_ref[...] = _dot(h, w0_ref[...]) * d_scr[pl.ds(i * BM, BM), :]


def _projmm(degp, x, wi, bi, w0):
    return pl.pallas_call(
        _projmm_body,
        grid=(NBM,),
        in_specs=[
            pl.BlockSpec((NW, NPAD), lambda i: (0, 0)),
            pl.BlockSpec((BM, D_IN), lambda i: (i, 0)),
            pl.BlockSpec((D_IN, DH), lambda i: (0, 0)),
            pl.BlockSpec((1, DH), lambda i: (0, 0)),
            pl.BlockSpec((DH, DH), lambda i: (0, 0)),
        ],
        out_specs=[
            pl.BlockSpec((BM, DH), lambda i: (i, 0)),
            pl.BlockSpec((BM, DH), lambda i: (i, 0)),
            pl.BlockSpec((NPAD, 1), lambda i: (0, 0)),
        ],
        out_shape=[
            jax.ShapeDtypeStruct((N, DH), jnp.float32),
            jax.ShapeDtypeStruct((N, DH), jnp.float32),
            jax.ShapeDtypeStruct((NPAD, 1), jnp.float32),
        ],
        scratch_shapes=[pltpu.VMEM((NPAD, 1), jnp.float32)],
    )(degp, x, wi, bi, w0)


def _bn_phase0(agg_ref, hws_ref, d_ref, b_ref, i, t_scr, s_scr, ss_scr):
    t = d_ref[...] * (agg_ref[0] + agg_ref[1] + 2.0 * hws_ref[...]) + b_ref[...]
    t_scr[pl.ds(i * BM, BM), :] = t
    ps = jnp.sum(t, axis=0, keepdims=True)
    pss = jnp.sum(t * t, axis=0, keepdims=True)

    @pl.when(i == 0)
    def _():
        s_scr[0:1, :] = ps
        ss_scr[0:1, :] = pss

    @pl.when(i > 0)
    def _():
        s_scr[0:1, :] += ps
        ss_scr[0:1, :] += pss


def _bn_phase1(g_ref, be_ref, h_ref, i, t_scr, s_scr, ss_scr):
    m = s_scr[0:1, :] / N
    v = ss_scr[0:1, :] / N - m * m
    t = t_scr[pl.ds(i * BM, BM), :]
    bn = (t - m) * lax.rsqrt(v + 1e-5) * g_ref[...] + be_ref[...]
    return jnp.maximum(bn, 0.0) + h_ref[...]


def _postmm_body(agg_ref, hws_ref, d_ref, b_ref, g_ref, be_ref, h_ref, w_ref,
                 oh_ref, ohws_ref, t_scr, s_scr, ss_scr):
    p = pl.program_id(0)
    i = pl.program_id(1)

    @pl.when(p == 0)
    def _():
        _bn_phase0(agg_ref, hws_ref, d_ref, b_ref, i, t_scr, s_scr, ss_scr)

    @pl.when(p == 1)
    def _():
        hn = _bn_phase1(g_ref, be_ref, h_ref, i, t_scr, s_scr, ss_scr)
        oh_ref[...] = hn
        ohws_ref[...] = _dot(hn, w_ref[...]) * d_ref[...]


def _postmm(agg, hws, dinv, b, g, be, h, w_next):
    return pl.pallas_call(
        _postmm_body,
        grid=(2, NBM),
        in_specs=[
            pl.BlockSpec((NC, BM, DH), lambda p, i: (0, (1 - p) * i, 0)),
            pl.BlockSpec((BM, DH), lambda p, i: ((1 - p) * i, 0)),
            pl.BlockSpec((BM, 1), lambda p, i: (i, 0)),
            pl.BlockSpec((1, DH), lambda p, i: (0, 0)),
            pl.BlockSpec((1, DH), lambda p, i: (0, 0)),
            pl.BlockSpec((1, DH), lambda p, i: (0, 0)),
            pl.BlockSpec((BM, DH), lambda p, i: (p * i, 0)),
            pl.BlockSpec((DH, DH), lambda p, i: (0, 0)),
        ],
        out_specs=[
            pl.BlockSpec((BM, DH), lambda p, i: (p * i, 0)),
            pl.BlockSpec((BM, DH), lambda p, i: (p * i, 0)),
        ],
        out_shape=[
            jax.ShapeDtypeStruct((N, DH), jnp.float32),
            jax.ShapeDtypeStruct((N, DH), jnp.float32),
        ],
        scratch_shapes=[
            pltpu.VMEM((N, DH), jnp.float32),
            pltpu.VMEM((8, DH), jnp.float32),
            pltpu.VMEM((8, DH), jnp.float32),
        ],
    )(agg, hws, dinv, b, g, be, h, w_next)


def _posttail_body(agg_ref, hws_ref, d_ref, b_ref, g_ref, be_ref, h_ref,
                   b3_ref, f1w_ref, f1b_ref, f2w_ref, f2b_ref, o_ref,
                   t_scr, s_scr, ss_scr, ps_scr, ct_scr):
    p = pl.program_id(0)
    i = pl.program_id(1)

    @pl.when(p == 0)
    def _():
        _bn_phase0(agg_ref, hws_ref, d_ref, b_ref, i, t_scr, s_scr, ss_scr)

    @pl.when(p == 1)
    def _():
        hn = _bn_phase1(g_ref, be_ref, h_ref, i, t_scr, s_scr, ss_scr)
        bb = b3_ref[0]
        gid = lax.broadcasted_iota(jnp.int32, (NG, BM), 0)
        P = (jnp.broadcast_to(bb, (NG, BM)) == gid).astype(jnp.float32)
        part = _dot(P, hn)
        cnt = jnp.sum(P, axis=1, keepdims=True)

        @pl.when(i == 0)
        def _():
            ps_scr[...] = part
            ct_scr[...] = jnp.broadcast_to(cnt, (NG, DH))

        @pl.when(i > 0)
        def _():
            ps_scr[...] += part
            ct_scr[...] += jnp.broadcast_to(cnt, (NG, DH))

        @pl.when(i == NBM - 1)
        def _():
            pooled = ps_scr[...] / jnp.maximum(ct_scr[...], 1.0)
            o1 = jnp.maximum(_dot(pooled, f1w_ref[...]) + f1b_ref[...], 0.0)
            o_ref[...] = _dot(o1, f2w_ref[...]) + f2b_ref[...]


def _posttail(agg, hws, dinv, b, g, be, h, batch3, f1w, f1b, f2w, f2b):
    return pl.pallas_call(
        _posttail_body,
        grid=(2, NBM),
        in_specs=[
            pl.BlockSpec((NC, BM, DH), lambda p, i: (0, (1 - p) * i, 0)),
            pl.BlockSpec((BM, DH), lambda p, i: ((1 - p) * i, 0)),
            pl.BlockSpec((BM, 1), lambda p, i: (i, 0)),
            pl.BlockSpec((1, DH), lambda p, i: (0, 0)),
            pl.BlockSpec((1, DH), lambda p, i: (0, 0)),
            pl.BlockSpec((1, DH), lambda p, i: (0, 0)),
            pl.BlockSpec((BM, DH), lambda p, i: (p * i, 0)),
            pl.BlockSpec((1, 1, BM), lambda p, i: (p * i, 0, 0)),
            pl.BlockSpec((DH, DH), lambda p, i: (0, 0)),
            pl.BlockSpec((1, DH), lambda p, i: (0, 0)),
            pl.BlockSpec((DH, DOUT), lambda p, i: (0, 0)),
            pl.BlockSpec((1, DOUT), lambda p, i: (0, 0)),
        ],
        out_specs=pl.BlockSpec((NG, DOUT), lambda p, i: (0, 0)),
        out_shape=jax.ShapeDtypeStruct((NG, DOUT), jnp.float32),
        scratch_shapes=[
            pltpu.VMEM((N, DH), jnp.float32),
            pltpu.VMEM((8, DH), jnp.float32),
            pltpu.VMEM((8, DH), jnp.float32),
            pltpu.VMEM((NG, DH), jnp.float32),
            pltpu.VMEM((NG, DH), jnp.float32),
        ],
    )(agg, hws, dinv, b, g, be, h, batch3, f1w, f1b, f2w, f2b)


def kernel(x, edge_index, batch, W_in, b_in,
           W_c0, b_c0, gamma0, beta0,
           W_c1, b_c1, gamma1, beta1,
           W_c2, b_c2, gamma2, beta2,
           fc1_W, fc1_b, fc2_W, fc2_b):
    row = edge_index[0]
    col = edge_index[1]
    zeros_nd = jnp.zeros((N, DH), jnp.float32)

    degp = _deg_call(col)
    h, hws, dinvp = _projmm(degp, x, W_in, b_in.reshape(1, DH), W_c0)
    dinv = dinvp[:N]

    agg = _seg_call(hws, row, col, zeros_nd)
    h, hws = _postmm(agg, hws, dinv, b_c0.reshape(1, DH),
                     gamma0.reshape(1, DH), beta0.reshape(1, DH), h, W_c1)

    agg = _seg_call(hws, row, col, zeros_nd)
    h, hws = _postmm(agg, hws, dinv, b_c1.reshape(1, DH),
                     gamma1.reshape(1, DH), beta1.reshape(1, DH), h, W_c2)

    agg = _seg_call(hws, row, col, zeros_nd)
    batch3 = batch.reshape(NBM, 1, BM)
    out = _posttail(agg, hws, dinv, b_c2.reshape(1, DH),
                    gamma2.reshape(1, DH), beta2.reshape(1, DH), h, batch3,
                    fc1_W, fc1_b.reshape(1, DH), fc2_W,
                    fc2_b.reshape(1, DOUT))
    return out


# speedup vs baseline: 1.0776x; 1.0776x over previous
import jax
import jax.numpy as jnp
from jax import lax
from jax.experimental import pallas as pl
from jax.experimental.pallas import tpu as pltpu
from jax.experimental.pallas import tpu_sc as plsc

N = 10000
E = 320000
DH = 128
D_IN = 128
NG = 64
DOUT = 16

NC = 2
NS = 16
NW = NC * NS
EPT = E // NW
CB = 40
NCHUNK = EPT // CB
G = 5
NGRP = NCHUNK // G
W16 = 624

HHALF = 5120
NPAD = 10112
HCH = EPT // 16

BM = 2000
NBM = N // BM

_mesh = plsc.VectorSubcoreMesh(core_axis_name="c", subcore_axis_name="s",
                               num_cores=NC, num_subcores=NS)


def _deg_body(colr, out, cidx, hist, red):
    c = lax.axis_index("c")
    s = lax.axis_index("s")
    wid = c * NS + s
    pltpu.sync_copy(colr.at[pl.ds(wid * EPT, EPT)], cidx)

    lane = lax.broadcasted_iota(jnp.int32, (16,), 0)
    lanebase = lane * HHALF
    onev = jnp.full((16,), 1.0, jnp.float32)
    zerov = jnp.full((16,), 0.0, jnp.float32)

    for p in range(2):
        base = p * HHALF

        def zbody(i, carry):
            for u in range(8):
                hist[pl.ds((i * 8 + u) * 16, 16)] = zerov
            return carry

        lax.fori_loop(0, 16 * HHALF // (16 * 8), zbody, 0)

        def sbody(i, carry):
            for u in range(5):
                k = i * 5 + u
                colv = cidx[pl.ds(k * 16, 16)]
                cshift = colv - base
                m = (cshift >= 0) & (cshift < HHALF)
                plsc.addupdate_scatter(hist, [lanebase + cshift], onev,
                                       mask=m)
            return carry

        lax.fori_loop(0, HCH // 5, sbody, 0)

        nred = (HHALF if p == 0 else NPAD - HHALF) // 16

        def rbody(i, carry):
            acc = hist[pl.ds(i * 16, 16)]
            for j in range(1, 16):
                acc = acc + hist[pl.ds(j * HHALF + i * 16, 16)]
            red[pl.ds(base + i * 16, 16)] = acc
            return carry

        lax.fori_loop(0, nred, rbody, 0)

    pltpu.sync_copy(red, out.at[wid])


_deg_call = pl.kernel(
    _deg_body,
    out_type=jax.ShapeDtypeStruct((NW, NPAD), jnp.float32),
    mesh=_mesh,
    scratch_types=[
        pltpu.VMEM((EPT,), jnp.int32),
        pltpu.VMEM((16 * HHALF,), jnp.float32),
        pltpu.VMEM((NPAD,), jnp.float32),
    ],
    compiler_params=pltpu.CompilerParams(needs_layout_passes=False),
)


def _seg_body(hws, rowr, colr, zeros, out, ridx, cidx,
              r0, r1, r2, r3, r4, agg,
              g0, g1, g2, g3, g4, s0, s1, s2, s3, s4):
    rows = [r0, r1, r2, r3, r4]
    gsem = [g0, g1, g2, g3, g4]
    ssem = [s0, s1, s2, s3, s4]
    c = lax.axis_index("c")
    s = lax.axis_index("s")
    wid = c * NS + s
    pltpu.sync_copy(rowr.at[pl.ds(wid * EPT, EPT)], ridx)
    pltpu.sync_copy(colr.at[pl.ds(wid * EPT, EPT)], cidx)

    @pl.when(s < NS - 1)
    def _():
        pltpu.sync_copy(zeros.at[pl.ds(s * W16, W16)],
                        agg.at[pl.ds(s * W16, W16)])

    @pl.when(s == NS - 1)
    def _():
        pltpu.sync_copy(zeros.at[pl.ds((NS - 1) * W16, N - (NS - 1) * W16)],
                        agg.at[pl.ds((NS - 1) * W16, N - (NS - 1) * W16)])

    plsc.subcore_barrier()

    for j in range(G):
        pltpu.async_copy(hws.at[ridx.at[pl.ds(j * CB, CB)]], rows[j], gsem[j])

    def grp(g, carry):
        for j in range(G):
            k = g * G + j
            pltpu.make_async_copy(hws.at[ridx.at[pl.ds(k * CB, CB)]],
                                  rows[j], gsem[j]).wait()
            pltpu.async_copy(rows[j], agg.at[cidx.at[pl.ds(k * CB, CB)]],
                             ssem[j], add=True)

        @pl.when(g < NGRP - 1)
        def _():
            for j in range(G):
                k = g * G + j
                pltpu.make_async_copy(rows[j],
                                      agg.at[cidx.at[pl.ds(k * CB, CB)]],
                                      ssem[j]).wait()
                pltpu.async_copy(hws.at[ridx.at[pl.ds((k + G) * CB, CB)]],
                                 rows[j], gsem[j])

        return carry

    lax.fori_loop(0, NGRP, grp, 0)
    for j in range(G):
        pltpu.make_async_copy(rows[j], agg.at[cidx.at[pl.ds(0, CB)]],
                              ssem[j]).wait()
    plsc.subcore_barrier()

    @pl.when(s < NS - 1)
    def _():
        pltpu.sync_copy(agg.at[pl.ds(s * W16, W16)],
                        out.at[c, pl.ds(s * W16, W16)])

    @pl.when(s == NS - 1)
    def _():
        pltpu.sync_copy(agg.at[pl.ds((NS - 1) * W16, N - (NS - 1) * W16)],
                        out.at[c, pl.ds((NS - 1) * W16, N - (NS - 1) * W16)])


_seg_call = pl.kernel(
    _seg_body,
    out_type=jax.ShapeDtypeStruct((NC, N, DH), jnp.float32),
    mesh=_mesh,
    scratch_types=[
        pltpu.VMEM((EPT,), jnp.int32),
        pltpu.VMEM((EPT,), jnp.int32),
    ] + [pltpu.VMEM((CB, DH), jnp.float32) for _ in range(G)] + [
        pltpu.VMEM_SHARED((N, DH), jnp.float32),
    ] + [pltpu.SemaphoreType.DMA for _ in range(2 * G)],
)


def _dot(a, b):
    return lax.dot_general(
        a, b, (((1,), (0,)), ((), ())),
        preferred_element_type=jnp.float32,
        precision=lax.Precision.HIGHEST,
    )


def _proj_body(x_ref, wi_ref, bi_ref, oh_ref):
    oh_ref[...] = _dot(x_ref[...], wi_ref[...]) + bi_ref[...]


def _proj(x, wi, bi):
    return pl.pallas_call(
        _proj_body,
        grid=(NBM,),
        in_specs=[
            pl.BlockSpec((BM, D_IN), lambda i: (i, 0)),
            pl.BlockSpec((D_IN, DH), lambda i: (0, 0)),
            pl.BlockSpec((1, DH), lambda i: (0, 0)),
        ],
        out_specs=pl.BlockSpec((BM, DH), lambda i: (i, 0)),
        out_shape=jax.ShapeDtypeStruct((N, DH), jnp.float32),
    )(x, wi, bi)


def _scalemm_body(degp_ref, h_ref, w0_ref, ohws_ref, od_ref, d_scr):
    i = pl.program_id(0)

    @pl.when(i == 0)
    def _():
        ones = jnp.ones((NW, 1), jnp.float32)
        deg = lax.dot_general(
            degp_ref[...], ones, (((0,), (0,)), ((), ())),
            preferred_element_type=jnp.float32,
            precision=lax.Precision.HIGHEST,
        ) + 2.0
        d = lax.rsqrt(deg)
        d_scr[...] = d
        od_ref[...] = d

    ohws_ref[...] = _dot(h_ref[...], w0_ref[...]) * d_scr[pl.ds(i * BM, BM), :]


def _scalemm(degp, h, w0):
    return pl.pallas_call(
        _scalemm_body,
        grid=(NBM,),
        in_specs=[
            pl.BlockSpec((NW, NPAD), lambda i: (0, 0)),
            pl.BlockSpec((BM, DH), lambda i: (i, 0)),
            pl.BlockSpec((DH, DH), lambda i: (0, 0)),
        ],
        out_specs=[
            pl.BlockSpec((BM, DH), lambda i: (i, 0)),
            pl.BlockSpec((NPAD, 1), lambda i: (0, 0)),
        ],
        out_shape=[
            jax.ShapeDtypeStruct((N, DH), jnp.float32),
            jax.ShapeDtypeStruct((NPAD, 1), jnp.float32),
        ],
        scratch_shapes=[pltpu.VMEM((NPAD, 1), jnp.float32)],
    )(degp, h, w0)


def _bn_phase0(agg_ref, hws_ref, d_ref, b_ref, i, t_scr, s_scr, ss_scr):
    t = d_ref[...] * (agg_ref[0] + agg_ref[1] + 2.0 * hws_ref[...]) + b_ref[...]
    t_scr[pl.ds(i * BM, BM), :] = t
    ps = jnp.sum(t, axis=0, keepdims=True)
    pss = jnp.sum(t * t, axis=0, keepdims=True)

    @pl.when(i == 0)
    def _():
        s_scr[0:1, :] = ps
        ss_scr[0:1, :] = pss

    @pl.when(i > 0)
    def _():
        s_scr[0:1, :] += ps
        ss_scr[0:1, :] += pss


def _bn_phase1(g_ref, be_ref, h_ref, i, t_scr, s_scr, ss_scr):
    m = s_scr[0:1, :] / N
    v = ss_scr[0:1, :] / N - m * m
    t = t_scr[pl.ds(i * BM, BM), :]
    bn = (t - m) * lax.rsqrt(v + 1e-5) * g_ref[...] + be_ref[...]
    return jnp.maximum(bn, 0.0) + h_ref[...]


def _postmm_body(agg_ref, hws_ref, d_ref, b_ref, g_ref, be_ref, h_ref, w_ref,
                 oh_ref, ohws_ref, t_scr, s_scr, ss_scr):
    p = pl.program_id(0)
    i = pl.program_id(1)

    @pl.when(p == 0)
    def _():
        _bn_phase0(agg_ref, hws_ref, d_ref, b_ref, i, t_scr, s_scr, ss_scr)

    @pl.when(p == 1)
    def _():
        hn = _bn_phase1(g_ref, be_ref, h_ref, i, t_scr, s_scr, ss_scr)
        oh_ref[...] = hn
        ohws_ref[...] = _dot(hn, w_ref[...]) * d_ref[...]


def _postmm(agg, hws, dinv, b, g, be, h, w_next):
    return pl.pallas_call(
        _postmm_body,
        grid=(2, NBM),
        in_specs=[
            pl.BlockSpec((NC, BM, DH), lambda p, i: (0, (1 - p) * i, 0)),
            pl.BlockSpec((BM, DH), lambda p, i: ((1 - p) * i, 0)),
            pl.BlockSpec((BM, 1), lambda p, i: (i, 0)),
            pl.BlockSpec((1, DH), lambda p, i: (0, 0)),
            pl.BlockSpec((1, DH), lambda p, i: (0, 0)),
            pl.BlockSpec((1, DH), lambda p, i: (0, 0)),
            pl.BlockSpec((BM, DH), lambda p, i: (p * i, 0)),
            pl.BlockSpec((DH, DH), lambda p, i: (0, 0)),
        ],
        out_specs=[
            pl.BlockSpec((BM, DH), lambda p, i: (p * i, 0)),
            pl.BlockSpec((BM, DH), lambda p, i: (p * i, 0)),
        ],
        out_shape=[
            jax.ShapeDtypeStruct((N, DH), jnp.float32),
            jax.ShapeDtypeStruct((N, DH), jnp.float32),
        ],
        scratch_shapes=[
            pltpu.VMEM((N, DH), jnp.float32),
            pltpu.VMEM((8, DH), jnp.float32),
            pltpu.VMEM((8, DH), jnp.float32),
        ],
    )(agg, hws, dinv, b, g, be, h, w_next)


def _posttail_body(agg_ref, hws_ref, d_ref, b_ref, g_ref, be_ref, h_ref,
                   b3_ref, f1w_ref, f1b_ref, f2w_ref, f2b_ref, o_ref,
                   t_scr, s_scr, ss_scr, ps_scr, ct_scr):
    p = pl.program_id(0)
    i = pl.program_id(1)

    @pl.when(p == 0)
    def _():
        _bn_phase0(agg_ref, hws_ref, d_ref, b_ref, i, t_scr, s_scr, ss_scr)

    @pl.when(p == 1)
    def _():
        hn = _bn_phase1(g_ref, be_ref, h_ref, i, t_scr, s_scr, ss_scr)
        bb = b3_ref[0]
        gid = lax.broadcasted_iota(jnp.int32, (NG, BM), 0)
        P = (jnp.broadcast_to(bb, (NG, BM)) == gid).astype(jnp.float32)
        part = _dot(P, hn)
        cnt = jnp.sum(P, axis=1, keepdims=True)

        @pl.when(i == 0)
        def _():
            ps_scr[...] = part
            ct_scr[...] = jnp.broadcast_to(cnt, (NG, DH))

        @pl.when(i > 0)
        def _():
            ps_scr[...] += part
            ct_scr[...] += jnp.broadcast_to(cnt, (NG, DH))

        @pl.when(i == NBM - 1)
        def _():
            pooled = ps_scr[...] / jnp.maximum(ct_scr[...], 1.0)
            o1 = jnp.maximum(_dot(pooled, f1w_ref[...]) + f1b_ref[...], 0.0)
            o_ref[...] = _dot(o1, f2w_ref[...]) + f2b_ref[...]


def _posttail(agg, hws, dinv, b, g, be, h, batch3, f1w, f1b, f2w, f2b):
    return pl.pallas_call(
        _posttail_body,
        grid=(2, NBM),
        in_specs=[
            pl.BlockSpec((NC, BM, DH), lambda p, i: (0, (1 - p) * i, 0)),
            pl.BlockSpec((BM, DH), lambda p, i: ((1 - p) * i, 0)),
            pl.BlockSpec((BM, 1), lambda p, i: (i, 0)),
            pl.BlockSpec((1, DH), lambda p, i: (0, 0)),
            pl.BlockSpec((1, DH), lambda p, i: (0, 0)),
            pl.BlockSpec((1, DH), lambda p, i: (0, 0)),
            pl.BlockSpec((BM, DH), lambda p, i: (p * i, 0)),
            pl.BlockSpec((1, 1, BM), lambda p, i: (p * i, 0, 0)),
            pl.BlockSpec((DH, DH), lambda p, i: (0, 0)),
            pl.BlockSpec((1, DH), lambda p, i: (0, 0)),
            pl.BlockSpec((DH, DOUT), lambda p, i: (0, 0)),
            pl.BlockSpec((1, DOUT), lambda p, i: (0, 0)),
        ],
        out_specs=pl.BlockSpec((NG, DOUT), lambda p, i: (0, 0)),
        out_shape=jax.ShapeDtypeStruct((NG, DOUT), jnp.float32),
        scratch_shapes=[
            pltpu.VMEM((N, DH), jnp.float32),
            pltpu.VMEM((8, DH), jnp.float32),
            pltpu.VMEM((8, DH), jnp.float32),
            pltpu.VMEM((NG, DH), jnp.float32),
            pltpu.VMEM((NG, DH), jnp.float32),
        ],
    )(agg, hws, dinv, b, g, be, h, batch3, f1w, f1b, f2w, f2b)


def kernel(x, edge_index, batch, W_in, b_in,
           W_c0, b_c0, gamma0, beta0,
           W_c1, b_c1, gamma1, beta1,
           W_c2, b_c2, gamma2, beta2,
           fc1_W, fc1_b, fc2_W, fc2_b):
    row = edge_index[0]
    col = edge_index[1]
    zeros_nd = jnp.zeros((N, DH), jnp.float32)

    degp = _deg_call(col)
    h = _proj(x, W_in, b_in.reshape(1, DH))
    hws, dinvp = _scalemm(degp, h, W_c0)
    dinv = dinvp[:N]

    agg = _seg_call(hws, row, col, zeros_nd)
    h, hws = _postmm(agg, hws, dinv, b_c0.reshape(1, DH),
                     gamma0.reshape(1, DH), beta0.reshape(1, DH), h, W_c1)

    agg = _seg_call(hws, row, col, zeros_nd)
    h, hws = _postmm(agg, hws, dinv, b_c1.reshape(1, DH),
                     gamma1.reshape(1, DH), beta1.reshape(1, DH), h, W_c2)

    agg = _seg_call(hws, row, col, zeros_nd)
    batch3 = batch.reshape(NBM, 1, BM)
    out = _posttail(agg, hws, dinv, b_c2.reshape(1, DH),
                    gamma2.reshape(1, DH), beta2.reshape(1, DH), h, batch3,
                    fc1_W, fc1_b.reshape(1, DH), fc2_W,
                    fc2_b.reshape(1, DOUT))
    return out


# speedup vs baseline: 1.0919x; 1.0133x over previous
import jax
import jax.numpy as jnp
from jax import lax
from jax.experimental import pallas as pl
from jax.experimental.pallas import tpu as pltpu
from jax.experimental.pallas import tpu_sc as plsc

N = 10000
E = 320000
DH = 128
D_IN = 128
NG = 64
DOUT = 16

NC = 2
NS = 16
NW = NC * NS
EPT = E // NW
CB = 40
NCHUNK = EPT // CB
G = 5
NGRP = NCHUNK // G
W16 = 624

HHALF = 5120
NPAD = 10112
HCH = EPT // 16

BM = 2000
NBM = N // BM

_mesh = plsc.VectorSubcoreMesh(core_axis_name="c", subcore_axis_name="s",
                               num_cores=NC, num_subcores=NS)


def _deg_body(colr, out, cidx, hist, red):
    c = lax.axis_index("c")
    s = lax.axis_index("s")
    wid = c * NS + s
    pltpu.sync_copy(colr.at[pl.ds(wid * EPT, EPT)], cidx)

    lane = lax.broadcasted_iota(jnp.int32, (16,), 0)
    lanebase = lane * HHALF
    onev = jnp.full((16,), 1.0, jnp.float32)
    zerov = jnp.full((16,), 0.0, jnp.float32)

    for p in range(2):
        base = p * HHALF

        def zbody(i, carry):
            for u in range(8):
                hist[pl.ds((i * 8 + u) * 16, 16)] = zerov
            return carry

        lax.fori_loop(0, 16 * HHALF // (16 * 8), zbody, 0)

        def sbody(i, carry):
            for u in range(5):
                k = i * 5 + u
                colv = cidx[pl.ds(k * 16, 16)]
                cshift = colv - base
                m = (cshift >= 0) & (cshift < HHALF)
                plsc.addupdate_scatter(hist, [lanebase + cshift], onev,
                                       mask=m)
            return carry

        lax.fori_loop(0, HCH // 5, sbody, 0)

        nred = (HHALF if p == 0 else NPAD - HHALF) // 16

        def rbody(i, carry):
            acc = hist[pl.ds(i * 16, 16)]
            for j in range(1, 16):
                acc = acc + hist[pl.ds(j * HHALF + i * 16, 16)]
            red[pl.ds(base + i * 16, 16)] = acc
            return carry

        lax.fori_loop(0, nred, rbody, 0)

    pltpu.sync_copy(red, out.at[wid])


_deg_call = pl.kernel(
    _deg_body,
    out_type=jax.ShapeDtypeStruct((NW, NPAD), jnp.float32),
    mesh=_mesh,
    scratch_types=[
        pltpu.VMEM((EPT,), jnp.int32),
        pltpu.VMEM((16 * HHALF,), jnp.float32),
        pltpu.VMEM((NPAD,), jnp.float32),
    ],
    compiler_params=pltpu.CompilerParams(needs_layout_passes=False),
)


def _seg_body(hws, rowr, colr, zeros, out, ridx, cidx,
              r0, r1, r2, r3, r4, agg,
              g0, g1, g2, g3, g4, s0, s1, s2, s3, s4):
    rows = [r0, r1, r2, r3, r4]
    gsem = [g0, g1, g2, g3, g4]
    ssem = [s0, s1, s2, s3, s4]
    c = lax.axis_index("c")
    s = lax.axis_index("s")
    wid = c * NS + s
    rcp = pltpu.make_async_copy(rowr.at[pl.ds(wid * EPT, EPT)], ridx, g0)
    rcp.start()
    ccp = pltpu.make_async_copy(colr.at[pl.ds(wid * EPT, EPT)], cidx, g1)
    ccp.start()

    @pl.when(s < NS - 1)
    def _():
        zcp = pltpu.make_async_copy(zeros.at[pl.ds(s * W16, W16)],
                                    agg.at[pl.ds(s * W16, W16)], g2)
        zcp.start()
        zcp.wait()

    @pl.when(s == NS - 1)
    def _():
        zcp = pltpu.make_async_copy(
            zeros.at[pl.ds((NS - 1) * W16, N - (NS - 1) * W16)],
            agg.at[pl.ds((NS - 1) * W16, N - (NS - 1) * W16)], g2)
        zcp.start()
        zcp.wait()

    rcp.wait()
    ccp.wait()
    plsc.subcore_barrier()

    for j in range(G):
        pltpu.async_copy(hws.at[ridx.at[pl.ds(j * CB, CB)]], rows[j], gsem[j])

    def grp(g, carry):
        for j in range(G):
            k = g * G + j
            pltpu.make_async_copy(hws.at[ridx.at[pl.ds(k * CB, CB)]],
                                  rows[j], gsem[j]).wait()
            pltpu.async_copy(rows[j], agg.at[cidx.at[pl.ds(k * CB, CB)]],
                             ssem[j], add=True)

        @pl.when(g < NGRP - 1)
        def _():
            for j in range(G):
                k = g * G + j
                pltpu.make_async_copy(rows[j],
                                      agg.at[cidx.at[pl.ds(k * CB, CB)]],
                                      ssem[j]).wait()
                pltpu.async_copy(hws.at[ridx.at[pl.ds((k + G) * CB, CB)]],
                                 rows[j], gsem[j])

        return carry

    lax.fori_loop(0, NGRP, grp, 0)
    for j in range(G):
        pltpu.make_async_copy(rows[j], agg.at[cidx.at[pl.ds(0, CB)]],
                              ssem[j]).wait()
    plsc.subcore_barrier()

    @pl.when(s < NS - 1)
    def _():
        pltpu.sync_copy(agg.at[pl.ds(s * W16, W16)],
                        out.at[c, pl.ds(s * W16, W16)])

    @pl.when(s == NS - 1)
    def _():
        pltpu.sync_copy(agg.at[pl.ds((NS - 1) * W16, N - (NS - 1) * W16)],
                        out.at[c, pl.ds((NS - 1) * W16, N - (NS - 1) * W16)])


_seg_call = pl.kernel(
    _seg_body,
    out_type=jax.ShapeDtypeStruct((NC, N, DH), jnp.float32),
    mesh=_mesh,
    scratch_types=[
        pltpu.VMEM((EPT,), jnp.int32),
        pltpu.VMEM((EPT,), jnp.int32),
    ] + [pltpu.VMEM((CB, DH), jnp.float32) for _ in range(G)] + [
        pltpu.VMEM_SHARED((N, DH), jnp.float32),
    ] + [pltpu.SemaphoreType.DMA for _ in range(2 * G)],
)


def _dot(a, b):
    return lax.dot_general(
        a, b, (((1,), (0,)), ((), ())),
        preferred_element_type=jnp.float32,
        precision=lax.Precision.HIGHEST,
    )


def _proj_body(x_ref, wi_ref, bi_ref, oh_ref):
    oh_ref[...] = _dot(x_ref[...], wi_ref[...]) + bi_ref[...]


def _proj(x, wi, bi):
    return pl.pallas_call(
        _proj_body,
        grid=(NBM,),
        in_specs=[
            pl.BlockSpec((BM, D_IN), lambda i: (i, 0)),
            pl.BlockSpec((D_IN, DH), lambda i: (0, 0)),
            pl.BlockSpec((1, DH), lambda i: (0, 0)),
        ],
        out_specs=pl.BlockSpec((BM, DH), lambda i: (i, 0)),
        out_shape=jax.ShapeDtypeStruct((N, DH), jnp.float32),
    )(x, wi, bi)


def _scalemm_body(degp_ref, h_ref, w0_ref, ohws_ref, od_ref, d_scr):
    i = pl.program_id(0)

    @pl.when(i == 0)
    def _():
        ones = jnp.ones((NW, 1), jnp.float32)
        deg = lax.dot_general(
            degp_ref[...], ones, (((0,), (0,)), ((), ())),
            preferred_element_type=jnp.float32,
            precision=lax.Precision.HIGHEST,
        ) + 2.0
        d = lax.rsqrt(deg)
        d_scr[...] = d
        od_ref[...] = d

    ohws_ref[...] = _dot(h_ref[...], w0_ref[...]) * d_scr[pl.ds(i * BM, BM), :]


def _scalemm(degp, h, w0):
    return pl.pallas_call(
        _scalemm_body,
        grid=(NBM,),
        in_specs=[
            pl.BlockSpec((NW, NPAD), lambda i: (0, 0)),
            pl.BlockSpec((BM, DH), lambda i: (i, 0)),
            pl.BlockSpec((DH, DH), lambda i: (0, 0)),
        ],
        out_specs=[
            pl.BlockSpec((BM, DH), lambda i: (i, 0)),
            pl.BlockSpec((NPAD, 1), lambda i: (0, 0)),
        ],
        out_shape=[
            jax.ShapeDtypeStruct((N, DH), jnp.float32),
            jax.ShapeDtypeStruct((NPAD, 1), jnp.float32),
        ],
        scratch_shapes=[pltpu.VMEM((NPAD, 1), jnp.float32)],
    )(degp, h, w0)


def _bn_phase0(agg_ref, hws_ref, d_ref, b_ref, i, t_scr, s_scr, ss_scr):
    t = d_ref[...] * (agg_ref[0] + agg_ref[1] + 2.0 * hws_ref[...]) + b_ref[...]
    t_scr[pl.ds(i * BM, BM), :] = t
    ps = jnp.sum(t, axis=0, keepdims=True)
    pss = jnp.sum(t * t, axis=0, keepdims=True)

    @pl.when(i == 0)
    def _():
        s_scr[0:1, :] = ps
        ss_scr[0:1, :] = pss

    @pl.when(i > 0)
    def _():
        s_scr[0:1, :] += ps
        ss_scr[0:1, :] += pss


def _bn_phase1(g_ref, be_ref, h_ref, i, t_scr, s_scr, ss_scr):
    m = s_scr[0:1, :] / N
    v = ss_scr[0:1, :] / N - m * m
    t = t_scr[pl.ds(i * BM, BM), :]
    bn = (t - m) * lax.rsqrt(v + 1e-5) * g_ref[...] + be_ref[...]
    return jnp.maximum(bn, 0.0) + h_ref[...]


def _postmm_body(agg_ref, hws_ref, d_ref, b_ref, g_ref, be_ref, h_ref, w_ref,
                 oh_ref, ohws_ref, t_scr, s_scr, ss_scr):
    p = pl.program_id(0)
    i = pl.program_id(1)

    @pl.when(p == 0)
    def _():
        _bn_phase0(agg_ref, hws_ref, d_ref, b_ref, i, t_scr, s_scr, ss_scr)

    @pl.when(p == 1)
    def _():
        hn = _bn_phase1(g_ref, be_ref, h_ref, i, t_scr, s_scr, ss_scr)
        oh_ref[...] = hn
        ohws_ref[...] = _dot(hn, w_ref[...]) * d_ref[...]


def _postmm(agg, hws, dinv, b, g, be, h, w_next):
    return pl.pallas_call(
        _postmm_body,
        grid=(2, NBM),
        in_specs=[
            pl.BlockSpec((NC, BM, DH), lambda p, i: (0, (1 - p) * i, 0)),
            pl.BlockSpec((BM, DH), lambda p, i: ((1 - p) * i, 0)),
            pl.BlockSpec((BM, 1), lambda p, i: (i, 0)),
            pl.BlockSpec((1, DH), lambda p, i: (0, 0)),
            pl.BlockSpec((1, DH), lambda p, i: (0, 0)),
            pl.BlockSpec((1, DH), lambda p, i: (0, 0)),
            pl.BlockSpec((BM, DH), lambda p, i: (p * i, 0)),
            pl.BlockSpec((DH, DH), lambda p, i: (0, 0)),
        ],
        out_specs=[
            pl.BlockSpec((BM, DH), lambda p, i: (p * i, 0)),
            pl.BlockSpec((BM, DH), lambda p, i: (p * i, 0)),
        ],
        out_shape=[
            jax.ShapeDtypeStruct((N, DH), jnp.float32),
            jax.ShapeDtypeStruct((N, DH), jnp.float32),
        ],
        scratch_shapes=[
            pltpu.VMEM((N, DH), jnp.float32),
            pltpu.VMEM((8, DH), jnp.float32),
            pltpu.VMEM((8, DH), jnp.float32),
        ],
    )(agg, hws, dinv, b, g, be, h, w_next)


def _posttail_body(agg_ref, hws_ref, d_ref, b_ref, g_ref, be_ref, h_ref,
                   b3_ref, f1w_ref, f1b_ref, f2w_ref, f2b_ref, o_ref,
                   t_scr, s_scr, ss_scr, ps_scr, ct_scr):
    p = pl.program_id(0)
    i = pl.program_id(1)

    @pl.when(p == 0)
    def _():
        _bn_phase0(agg_ref, hws_ref, d_ref, b_ref, i, t_scr, s_scr, ss_scr)

    @pl.when(p == 1)
    def _():
        hn = _bn_phase1(g_ref, be_ref, h_ref, i, t_scr, s_scr, ss_scr)
        bb = b3_ref[0]
        gid = lax.broadcasted_iota(jnp.int32, (NG, BM), 0)
        P = (jnp.broadcast_to(bb, (NG, BM)) == gid).astype(jnp.float32)
        part = _dot(P, hn)
        cnt = jnp.sum(P, axis=1, keepdims=True)

        @pl.when(i == 0)
        def _():
            ps_scr[...] = part
            ct_scr[...] = jnp.broadcast_to(cnt, (NG, DH))

        @pl.when(i > 0)
        def _():
            ps_scr[...] += part
            ct_scr[...] += jnp.broadcast_to(cnt, (NG, DH))

        @pl.when(i == NBM - 1)
        def _():
            pooled = ps_scr[...] / jnp.maximum(ct_scr[...], 1.0)
            o1 = jnp.maximum(_dot(pooled, f1w_ref[...]) + f1b_ref[...], 0.0)
            o_ref[...] = _dot(o1, f2w_ref[...]) + f2b_ref[...]


def _posttail(agg, hws, dinv, b, g, be, h, batch3, f1w, f1b, f2w, f2b):
    return pl.pallas_call(
        _posttail_body,
        grid=(2, NBM),
        in_specs=[
            pl.BlockSpec((NC, BM, DH), lambda p, i: (0, (1 - p) * i, 0)),
            pl.BlockSpec((BM, DH), lambda p, i: ((1 - p) * i, 0)),
            pl.BlockSpec((BM, 1), lambda p, i: (i, 0)),
            pl.BlockSpec((1, DH), lambda p, i: (0, 0)),
            pl.BlockSpec((1, DH), lambda p, i: (0, 0)),
            pl.BlockSpec((1, DH), lambda p, i: (0, 0)),
            pl.BlockSpec((BM, DH), lambda p, i: (p * i, 0)),
            pl.BlockSpec((1, 1, BM), lambda p, i: (p * i, 0, 0)),
            pl.BlockSpec((DH, DH), lambda p, i: (0, 0)),
            pl.BlockSpec((1, DH), lambda p, i: (0, 0)),
            pl.BlockSpec((DH, DOUT), lambda p, i: (0, 0)),
            pl.BlockSpec((1, DOUT), lambda p, i: (0, 0)),
        ],
        out_specs=pl.BlockSpec((NG, DOUT), lambda p, i: (0, 0)),
        out_shape=jax.ShapeDtypeStruct((NG, DOUT), jnp.float32),
        scratch_shapes=[
            pltpu.VMEM((N, DH), jnp.float32),
            pltpu.VMEM((8, DH), jnp.float32),
            pltpu.VMEM((8, DH), jnp.float32),
            pltpu.VMEM((NG, DH), jnp.float32),
            pltpu.VMEM((NG, DH), jnp.float32),
        ],
    )(agg, hws, dinv, b, g, be, h, batch3, f1w, f1b, f2w, f2b)


def kernel(x, edge_index, batch, W_in, b_in,
           W_c0, b_c0, gamma0, beta0,
           W_c1, b_c1, gamma1, beta1,
           W_c2, b_c2, gamma2, beta2,
           fc1_W, fc1_b, fc2_W, fc2_b):
    row = edge_index[0]
    col = edge_index[1]
    zeros_nd = jnp.zeros((N, DH), jnp.float32)

    degp = _deg_call(col)
    h = _proj(x, W_in, b_in.reshape(1, DH))
    hws, dinvp = _scalemm(degp, h, W_c0)
    dinv = dinvp[:N]

    agg = _seg_call(hws, row, col, zeros_nd)
    h, hws = _postmm(agg, hws, dinv, b_c0.reshape(1, DH),
                     gamma0.reshape(1, DH), beta0.reshape(1, DH), h, W_c1)

    agg = _seg_call(hws, row, col, zeros_nd)
    h, hws = _postmm(agg, hws, dinv, b_c1.reshape(1, DH),
                     gamma1.reshape(1, DH), beta1.reshape(1, DH), h, W_c2)

    agg = _seg_call(hws, row, col, zeros_nd)
    batch3 = batch.reshape(NBM, 1, BM)
    out = _posttail(agg, hws, dinv, b_c2.reshape(1, DH),
                    gamma2.reshape(1, DH), beta2.reshape(1, DH), h, batch3,
                    fc1_W, fc1_b.reshape(1, DH), fc2_W,
                    fc2_b.reshape(1, DOUT))
    return out


# speedup vs baseline: 1.0933x; 1.0013x over previous
import jax
import jax.numpy as jnp
from jax import lax
from jax.experimental import pallas as pl
from jax.experimental.pallas import tpu as pltpu
from jax.experimental.pallas import tpu_sc as plsc

N = 10000
E = 320000
DH = 128
D_IN = 128
NG = 64
DOUT = 16

NC = 2
NS = 16
NW = NC * NS
EPT = E // NW
CB = 40
NCHUNK = EPT // CB
G = 5
NGRP = NCHUNK // G
W16 = 624

HHALF = 5120
NPAD = 10112
HCH = EPT // 16

BM = 2000
NBM = N // BM

_mesh = plsc.VectorSubcoreMesh(core_axis_name="c", subcore_axis_name="s",
                               num_cores=NC, num_subcores=NS)


def _deg_body(colr, out, cidx, hist, red, isem):
    c = lax.axis_index("c")
    s = lax.axis_index("s")
    wid = c * NS + s
    icp = pltpu.make_async_copy(colr.at[pl.ds(wid * EPT, EPT)], cidx, isem)
    icp.start()

    lane = lax.broadcasted_iota(jnp.int32, (16,), 0)
    lanebase = lane * HHALF
    onev = jnp.full((16,), 1.0, jnp.float32)
    zerov = jnp.full((16,), 0.0, jnp.float32)

    def _zero_hist():
        def zbody(i, carry):
            for u in range(8):
                hist[pl.ds((i * 8 + u) * 16, 16)] = zerov
            return carry

        lax.fori_loop(0, 16 * HHALF // (16 * 8), zbody, 0)

    _zero_hist()
    icp.wait()

    for p in range(2):
        base = p * HHALF
        if p == 1:
            _zero_hist()

        def sbody(i, carry):
            for u in range(5):
                k = i * 5 + u
                colv = cidx[pl.ds(k * 16, 16)]
                cshift = colv - base
                m = (cshift >= 0) & (cshift < HHALF)
                plsc.addupdate_scatter(hist, [lanebase + cshift], onev,
                                       mask=m)
            return carry

        lax.fori_loop(0, HCH // 5, sbody, 0)

        nred = (HHALF if p == 0 else NPAD - HHALF) // 16

        def rbody(i, carry):
            acc = hist[pl.ds(i * 16, 16)]
            for j in range(1, 16):
                acc = acc + hist[pl.ds(j * HHALF + i * 16, 16)]
            red[pl.ds(base + i * 16, 16)] = acc
            return carry

        lax.fori_loop(0, nred, rbody, 0)

    pltpu.sync_copy(red, out.at[wid])


_deg_call = pl.kernel(
    _deg_body,
    out_type=jax.ShapeDtypeStruct((NW, NPAD), jnp.float32),
    mesh=_mesh,
    scratch_types=[
        pltpu.VMEM((EPT,), jnp.int32),
        pltpu.VMEM((16 * HHALF,), jnp.float32),
        pltpu.VMEM((NPAD,), jnp.float32),
        pltpu.SemaphoreType.DMA,
    ],
    compiler_params=pltpu.CompilerParams(needs_layout_passes=False),
)


def _seg_body(hws, rowr, colr, zeros, out, ridx, cidx,
              r0, r1, r2, r3, r4, agg,
              g0, g1, g2, g3, g4, s0, s1, s2, s3, s4):
    rows = [r0, r1, r2, r3, r4]
    gsem = [g0, g1, g2, g3, g4]
    ssem = [s0, s1, s2, s3, s4]
    c = lax.axis_index("c")
    s = lax.axis_index("s")
    wid = c * NS + s
    rcp = pltpu.make_async_copy(rowr.at[pl.ds(wid * EPT, EPT)], ridx, g0)
    rcp.start()
    ccp = pltpu.make_async_copy(colr.at[pl.ds(wid * EPT, EPT)], cidx, g1)
    ccp.start()

    @pl.when(s < NS - 1)
    def _():
        zcp = pltpu.make_async_copy(zeros.at[pl.ds(s * W16, W16)],
                                    agg.at[pl.ds(s * W16, W16)], g2)
        zcp.start()
        zcp.wait()

    @pl.when(s == NS - 1)
    def _():
        zcp = pltpu.make_async_copy(
            zeros.at[pl.ds((NS - 1) * W16, N - (NS - 1) * W16)],
            agg.at[pl.ds((NS - 1) * W16, N - (NS - 1) * W16)], g2)
        zcp.start()
        zcp.wait()

    rcp.wait()
    ccp.wait()
    plsc.subcore_barrier()

    for j in range(G):
        pltpu.async_copy(hws.at[ridx.at[pl.ds(j * CB, CB)]], rows[j], gsem[j])

    def grp(g, carry):
        for j in range(G):
            k = g * G + j
            pltpu.make_async_copy(hws.at[ridx.at[pl.ds(k * CB, CB)]],
                                  rows[j], gsem[j]).wait()
            pltpu.async_copy(rows[j], agg.at[cidx.at[pl.ds(k * CB, CB)]],
                             ssem[j], add=True)

        @pl.when(g < NGRP - 1)
        def _():
            for j in range(G):
                k = g * G + j
                pltpu.make_async_copy(rows[j],
                                      agg.at[cidx.at[pl.ds(k * CB, CB)]],
                                      ssem[j]).wait()
                pltpu.async_copy(hws.at[ridx.at[pl.ds((k + G) * CB, CB)]],
                                 rows[j], gsem[j])

        return carry

    lax.fori_loop(0, NGRP, grp, 0)
    for j in range(G):
        pltpu.make_async_copy(rows[j], agg.at[cidx.at[pl.ds(0, CB)]],
                              ssem[j]).wait()
    plsc.subcore_barrier()

    @pl.when(s < NS - 1)
    def _():
        pltpu.sync_copy(agg.at[pl.ds(s * W16, W16)],
                        out.at[c, pl.ds(s * W16, W16)])

    @pl.when(s == NS - 1)
    def _():
        pltpu.sync_copy(agg.at[pl.ds((NS - 1) * W16, N - (NS - 1) * W16)],
                        out.at[c, pl.ds((NS - 1) * W16, N - (NS - 1) * W16)])


_seg_call = pl.kernel(
    _seg_body,
    out_type=jax.ShapeDtypeStruct((NC, N, DH), jnp.float32),
    mesh=_mesh,
    scratch_types=[
        pltpu.VMEM((EPT,), jnp.int32),
        pltpu.VMEM((EPT,), jnp.int32),
    ] + [pltpu.VMEM((CB, DH), jnp.float32) for _ in range(G)] + [
        pltpu.VMEM_SHARED((N, DH), jnp.float32),
    ] + [pltpu.SemaphoreType.DMA for _ in range(2 * G)],
)


def _dot(a, b):
    return lax.dot_general(
        a, b, (((1,), (0,)), ((), ())),
        preferred_element_type=jnp.float32,
        precision=lax.Precision.HIGHEST,
    )


def _proj_body(x_ref, wi_ref, bi_ref, oh_ref):
    oh_ref[...] = _dot(x_ref[...], wi_ref[...]) + bi_ref[...]


def _proj(x, wi, bi):
    return pl.pallas_call(
        _proj_body,
        grid=(NBM,),
        in_specs=[
            pl.BlockSpec((BM, D_IN), lambda i: (i, 0)),
            pl.BlockSpec((D_IN, DH), lambda i: (0, 0)),
            pl.BlockSpec((1, DH), lambda i: (0, 0)),
        ],
        out_specs=pl.BlockSpec((BM, DH), lambda i: (i, 0)),
        out_shape=jax.ShapeDtypeStruct((N, DH), jnp.float32),
    )(x, wi, bi)


def _scalemm_body(degp_ref, h_ref, w0_ref, ohws_ref, od_ref, d_scr):
    i = pl.program_id(0)

    @pl.when(i == 0)
    def _():
        ones = jnp.ones((NW, 1), jnp.float32)
        deg = lax.dot_general(
            degp_ref[...], ones, (((0,), (0,)), ((), ())),
            preferred_element_type=jnp.float32,
            precision=lax.Precision.HIGHEST,
        ) + 2.0
        d = lax.rsqrt(deg)
        d_scr[...] = d
        od_ref[...] = d

    ohws_ref[...] = _dot(h_ref[...], w0_ref[...]) * d_scr[pl.ds(i * BM, BM), :]


def _scalemm(degp, h, w0):
    return pl.pallas_call(
        _scalemm_body,
        grid=(NBM,),
        in_specs=[
            pl.BlockSpec((NW, NPAD), lambda i: (0, 0)),
            pl.BlockSpec((BM, DH), lambda i: (i, 0)),
            pl.BlockSpec((DH, DH), lambda i: (0, 0)),
        ],
        out_specs=[
            pl.BlockSpec((BM, DH), lambda i: (i, 0)),
            pl.BlockSpec((NPAD, 1), lambda i: (0, 0)),
        ],
        out_shape=[
            jax.ShapeDtypeStruct((N, DH), jnp.float32),
            jax.ShapeDtypeStruct((NPAD, 1), jnp.float32),
        ],
        scratch_shapes=[pltpu.VMEM((NPAD, 1), jnp.float32)],
    )(degp, h, w0)


def _bn_phase0(agg_ref, hws_ref, d_ref, b_ref, i, t_scr, s_scr, ss_scr):
    t = d_ref[...] * (agg_ref[0] + agg_ref[1] + 2.0 * hws_ref[...]) + b_ref[...]
    t_scr[pl.ds(i * BM, BM), :] = t
    ps = jnp.sum(t, axis=0, keepdims=True)
    pss = jnp.sum(t * t, axis=0, keepdims=True)

    @pl.when(i == 0)
    def _():
        s_scr[0:1, :] = ps
        ss_scr[0:1, :] = pss

    @pl.when(i > 0)
    def _():
        s_scr[0:1, :] += ps
        ss_scr[0:1, :] += pss


def _bn_phase1(g_ref, be_ref, h_ref, i, t_scr, s_scr, ss_scr):
    m = s_scr[0:1, :] / N
    v = ss_scr[0:1, :] / N - m * m
    t = t_scr[pl.ds(i * BM, BM), :]
    bn = (t - m) * lax.rsqrt(v + 1e-5) * g_ref[...] + be_ref[...]
    return jnp.maximum(bn, 0.0) + h_ref[...]


def _postmm_body(agg_ref, hws_ref, d_ref, b_ref, g_ref, be_ref, h_ref, w_ref,
                 oh_ref, ohws_ref, t_scr, s_scr, ss_scr):
    p = pl.program_id(0)
    i = pl.program_id(1)

    @pl.when(p == 0)
    def _():
        _bn_phase0(agg_ref, hws_ref, d_ref, b_ref, i, t_scr, s_scr, ss_scr)

    @pl.when(p == 1)
    def _():
        hn = _bn_phase1(g_ref, be_ref, h_ref, i, t_scr, s_scr, ss_scr)
        oh_ref[...] = hn
        ohws_ref[...] = _dot(hn, w_ref[...]) * d_ref[...]


def _postmm(agg, hws, dinv, b, g, be, h, w_next):
    return pl.pallas_call(
        _postmm_body,
        grid=(2, NBM),
        in_specs=[
            pl.BlockSpec((NC, BM, DH), lambda p, i: (0, (1 - p) * i, 0)),
            pl.BlockSpec((BM, DH), lambda p, i: ((1 - p) * i, 0)),
            pl.BlockSpec((BM, 1), lambda p, i: (i, 0)),
            pl.BlockSpec((1, DH), lambda p, i: (0, 0)),
            pl.BlockSpec((1, DH), lambda p, i: (0, 0)),
            pl.BlockSpec((1, DH), lambda p, i: (0, 0)),
            pl.BlockSpec((BM, DH), lambda p, i: (p * i, 0)),
            pl.BlockSpec((DH, DH), lambda p, i: (0, 0)),
        ],
        out_specs=[
            pl.BlockSpec((BM, DH), lambda p, i: (p * i, 0)),
            pl.BlockSpec((BM, DH), lambda p, i: (p * i, 0)),
        ],
        out_shape=[
            jax.ShapeDtypeStruct((N, DH), jnp.float32),
            jax.ShapeDtypeStruct((N, DH), jnp.float32),
        ],
        scratch_shapes=[
            pltpu.VMEM((N, DH), jnp.float32),
            pltpu.VMEM((8, DH), jnp.float32),
            pltpu.VMEM((8, DH), jnp.float32),
        ],
    )(agg, hws, dinv, b, g, be, h, w_next)


def _posttail_body(agg_ref, hws_ref, d_ref, b_ref, g_ref, be_ref, h_ref,
                   b3_ref, f1w_ref, f1b_ref, f2w_ref, f2b_ref, o_ref,
                   t_scr, s_scr, ss_scr, ps_scr, ct_scr):
    p = pl.program_id(0)
    i = pl.program_id(1)

    @pl.when(p == 0)
    def _():
        _bn_phase0(agg_ref, hws_ref, d_ref, b_ref, i, t_scr, s_scr, ss_scr)

    @pl.when(p == 1)
    def _():
        hn = _bn_phase1(g_ref, be_ref, h_ref, i, t_scr, s_scr, ss_scr)
        bb = b3_ref[0]
        gid = lax.broadcasted_iota(jnp.int32, (NG, BM), 0)
        P = (jnp.broadcast_to(bb, (NG, BM)) == gid).astype(jnp.float32)
        part = _dot(P, hn)
        cnt = jnp.sum(P, axis=1, keepdims=True)

        @pl.when(i == 0)
        def _():
            ps_scr[...] = part
            ct_scr[...] = jnp.broadcast_to(cnt, (NG, DH))

        @pl.when(i > 0)
        def _():
            ps_scr[...] += part
            ct_scr[...] += jnp.broadcast_to(cnt, (NG, DH))

        @pl.when(i == NBM - 1)
        def _():
            pooled = ps_scr[...] / jnp.maximum(ct_scr[...], 1.0)
            o1 = jnp.maximum(_dot(pooled, f1w_ref[...]) + f1b_ref[...], 0.0)
            o_ref[...] = _dot(o1, f2w_ref[...]) + f2b_ref[...]


def _posttail(agg, hws, dinv, b, g, be, h, batch3, f1w, f1b, f2w, f2b):
    return pl.pallas_call(
        _posttail_body,
        grid=(2, NBM),
        in_specs=[
            pl.BlockSpec((NC, BM, DH), lambda p, i: (0, (1 - p) * i, 0)),
            pl.BlockSpec((BM, DH), lambda p, i: ((1 - p) * i, 0)),
            pl.BlockSpec((BM, 1), lambda p, i: (i, 0)),
            pl.BlockSpec((1, DH), lambda p, i: (0, 0)),
            pl.BlockSpec((1, DH), lambda p, i: (0, 0)),
            pl.BlockSpec((1, DH), lambda p, i: (0, 0)),
            pl.BlockSpec((BM, DH), lambda p, i: (p * i, 0)),
            pl.BlockSpec((1, 1, BM), lambda p, i: (p * i, 0, 0)),
            pl.BlockSpec((DH, DH), lambda p, i: (0, 0)),
            pl.BlockSpec((1, DH), lambda p, i: (0, 0)),
            pl.BlockSpec((DH, DOUT), lambda p, i: (0, 0)),
            pl.BlockSpec((1, DOUT), lambda p, i: (0, 0)),
        ],
        out_specs=pl.BlockSpec((NG, DOUT), lambda p, i: (0, 0)),
        out_shape=jax.ShapeDtypeStruct((NG, DOUT), jnp.float32),
        scratch_shapes=[
            pltpu.VMEM((N, DH), jnp.float32),
            pltpu.VMEM((8, DH), jnp.float32),
            pltpu.VMEM((8, DH), jnp.float32),
            pltpu.VMEM((NG, DH), jnp.float32),
            pltpu.VMEM((NG, DH), jnp.float32),
        ],
    )(agg, hws, dinv, b, g, be, h, batch3, f1w, f1b, f2w, f2b)


def kernel(x, edge_index, batch, W_in, b_in,
           W_c0, b_c0, gamma0, beta0,
           W_c1, b_c1, gamma1, beta1,
           W_c2, b_c2, gamma2, beta2,
           fc1_W, fc1_b, fc2_W, fc2_b):
    row = edge_index[0]
    col = edge_index[1]
    zeros_nd = jnp.zeros((N, DH), jnp.float32)

    degp = _deg_call(col)
    h = _proj(x, W_in, b_in.reshape(1, DH))
    hws, dinvp = _scalemm(degp, h, W_c0)
    dinv = dinvp[:N]

    agg = _seg_call(hws, row, col, zeros_nd)
    h, hws = _postmm(agg, hws, dinv, b_c0.reshape(1, DH),
                     gamma0.reshape(1, DH), beta0.reshape(1, DH), h, W_c1)

    agg = _seg_call(hws, row, col, zeros_nd)
    h, hws = _postmm(agg, hws, dinv, b_c1.reshape(1, DH),
                     gamma1.reshape(1, DH), beta1.reshape(1, DH), h, W_c2)

    agg = _seg_call(hws, row, col, zeros_nd)
    batch3 = batch.reshape(NBM, 1, BM)
    out = _posttail(agg, hws, dinv, b_c2.reshape(1, DH),
                    gamma2.reshape(1, DH), beta2.reshape(1, DH), h, batch3,
                    fc1_W, fc1_b.reshape(1, DH), fc2_W,
                    fc2_b.reshape(1, DOUT))
    return out
